# baseline (speedup 1.0000x reference)
import functools

import jax
import jax.numpy as jnp
from jax import lax
from jax.experimental import pallas as pl
from jax.experimental.pallas import tpu as pltpu
from jax.experimental.pallas import tpu_sc as plsc

VOCAB = 1000000
EMB_DIM = 64
NUM_Y = 2
BATCH = 4096
HIST = 200

NUM_TILES = 32
BAGS_PER_TILE = BATCH // NUM_TILES
CHUNK = 104
HALF = HIST // 2
LANES = 16
VREGS_PER_ROW = EMB_DIM // LANES


def _sc_body(idx_hbm, table_hbm, w_hbm, b_hbm, out_hbm,
             idx_v, rows_a, rows_b, w_v, b_v, logit_v, sem_a, sem_b):
    wid = lax.axis_index("s") * 2 + lax.axis_index("c")

    pltpu.sync_copy(idx_hbm.at[wid], idx_v)
    pltpu.sync_copy(w_hbm, w_v)
    pltpu.sync_copy(b_hbm, b_v)

    w_regs = [[w_v[c, pl.ds(k * LANES, LANES)] for k in range(VREGS_PER_ROW)]
              for c in range(NUM_Y)]
    b_reg = b_v[...]
    b_sc = [b_reg[c] for c in range(NUM_Y)]
    inv_n = jnp.float32(1.0 / HIST)
    lane_iota = lax.iota(jnp.int32, LANES)
    lane_mask = lane_iota < NUM_Y
    b_sel = jnp.where(lane_iota == 0, b_reg[0], b_reg[1])
    perms = [lane_iota ^ s for s in (8, 4, 2, 1)]

    def lane_sum(v):
        for p in perms:
            v = v + v.at[p].get(mode="promise_in_bounds")
        return v

    def gather_bag(bag, rows_ref, sem):
        cp0 = pltpu.async_copy(
            table_hbm.at[idx_v.at[2 * bag]], rows_ref.at[pl.ds(0, CHUNK)], sem)
        cp1 = pltpu.async_copy(
            table_hbm.at[idx_v.at[2 * bag + 1]],
            rows_ref.at[pl.ds(CHUNK, CHUNK)], sem)
        return cp0, cp1

    def reduce_bag(bag, rows_ref):
        def body(j, accs):
            return tuple(
                accs[k]
                + rows_ref[j, pl.ds(k * LANES, LANES)]
                + rows_ref[j + CHUNK, pl.ds(k * LANES, LANES)]
                for k in range(VREGS_PER_ROW))
        zeros = tuple(jnp.zeros((LANES,), jnp.float32)
                      for _ in range(VREGS_PER_ROW))
        accs = lax.fori_loop(0, HALF, body, zeros)
        pooled = [a * inv_n for a in accs]
        reds = []
        for c in range(NUM_Y):
            prod = pooled[0] * w_regs[c][0]
            for k in range(1, VREGS_PER_ROW):
                prod = prod + pooled[k] * w_regs[c][k]
            reds.append(lane_sum(prod))
        vals = jnp.where(lane_iota == 0, reds[0], reds[1]) + b_sel
        plsc.store_scatter(logit_v, [2 * bag + lane_iota], vals,
                           mask=lane_mask)

    def pair_body(g, carry):
        bag0 = 2 * g
        bag1 = 2 * g + 1
        cps_a = gather_bag(bag0, rows_a, sem_a)
        cps_b = gather_bag(bag1, rows_b, sem_b)
        cps_a[0].wait()
        cps_a[1].wait()
        reduce_bag(bag0, rows_a)
        cps_b[0].wait()
        cps_b[1].wait()
        reduce_bag(bag1, rows_b)
        return carry

    lax.fori_loop(0, BAGS_PER_TILE // 2, pair_body, 0)

    for i in range(2 * BAGS_PER_TILE // LANES):
        x = logit_v[pl.ds(i * LANES, LANES)]
        logit_v[pl.ds(i * LANES, LANES)] = 1.0 / (1.0 + jnp.exp(-x))
    pltpu.sync_copy(logit_v, out_hbm.at[pl.ds(wid * 2 * BAGS_PER_TILE,
                                              2 * BAGS_PER_TILE)])


@jax.jit
def _sc_call(idx, table, w, b_pad):
    run = functools.partial(
        pl.kernel,
        out_type=jax.ShapeDtypeStruct((BATCH * NUM_Y,), jnp.float32),
        mesh=plsc.VectorSubcoreMesh(core_axis_name="c", subcore_axis_name="s"),
        compiler_params=pltpu.CompilerParams(
            needs_layout_passes=False, use_tc_tiling_on_sc=False),
        scratch_types=[
            pltpu.VMEM((2 * BAGS_PER_TILE, CHUNK), jnp.int32),
            pltpu.VMEM((2 * CHUNK, EMB_DIM), jnp.float32),
            pltpu.VMEM((2 * CHUNK, EMB_DIM), jnp.float32),
            pltpu.VMEM((NUM_Y, EMB_DIM), jnp.float32),
            pltpu.VMEM((LANES,), jnp.float32),
            pltpu.VMEM((2 * BAGS_PER_TILE,), jnp.float32),
            pltpu.SemaphoreType.DMA,
            pltpu.SemaphoreType.DMA,
        ],
    )(_sc_body)
    return run(idx, table, w, b_pad)


def kernel(input, emb_weight, W, b):
    idx = input.astype(jnp.int32).reshape(NUM_TILES, 2 * BAGS_PER_TILE, HALF)
    idx = jnp.pad(idx, ((0, 0), (0, 0), (0, CHUNK - HALF)))
    b_pad = jnp.pad(b.astype(jnp.float32), (0, LANES - NUM_Y))
    out_flat = _sc_call(idx, emb_weight, W.astype(jnp.float32), b_pad)
    return out_flat.reshape(BATCH, NUM_Y)
